# baseline (speedup 1.0000x reference)
import functools

import jax
import jax.numpy as jnp
from jax import lax
from jax.experimental import pallas as pl
from jax.experimental.pallas import tpu as pltpu
from jax.experimental.pallas import tpu_sc as plsc

_K = 5532
_C = 256
_NPIX = 2048
_N_TOT = 4096

_NW = 32
_BPW = _N_TOT // _NW


def _lse_kernel(lut_ref, x_ref, out_ref):
    b = pl.program_id(0)

    x = x_ref[0].astype(jnp.bfloat16)
    w = lut_ref[...].astype(jnp.bfloat16)
    s_blk = jax.lax.dot_general(w, x, (((1,), (0,)), ((), ())),
                                preferred_element_type=jnp.float32)

    p = jnp.exp(s_blk)
    s = jnp.sum(p, axis=0, keepdims=True)

    part = jnp.sum(jnp.log(s), axis=1, keepdims=True) * (1.0 / _N_TOT)

    @pl.when(b == 0)
    def _w():
        out_ref[...] = part

    @pl.when(b > 0)
    def _a():
        out_ref[...] += part


def _lse_call(lut, x):
    return pl.pallas_call(
        _lse_kernel,
        grid=(2,),
        in_specs=[
            pl.BlockSpec((_K, _C), lambda b: (0, 0)),
            pl.BlockSpec((1, _C, _NPIX), lambda b: (b, 0, 0)),
        ],
        out_specs=pl.BlockSpec((1, 1), lambda b: (0, 0)),
        out_shape=jax.ShapeDtypeStruct((1, 1), jnp.float32),
        compiler_params=pltpu.CompilerParams(
            dimension_semantics=("arbitrary",),
        ),
    )(lut, x)


@functools.partial(
    pl.kernel,
    mesh=plsc.VectorSubcoreMesh(core_axis_name="c", subcore_axis_name="s"),
    out_type=jax.ShapeDtypeStruct((_NW, 16), jnp.float32),
    scratch_types=[
        pltpu.VMEM((_BPW,), jnp.int32),
        pltpu.VMEM((_BPW, _C), jnp.float32),
        pltpu.VMEM((_BPW, _C), jnp.float32),
        pltpu.VMEM((16,), jnp.float32),
        pltpu.SemaphoreType.DMA,
    ],
)
def _tgt_dot_kernel(lut_hbm, xt_hbm, tgt_hbm, out_hbm,
                    idx_v, rows_v, xv, acc_v, sem):
    wid = lax.axis_index("s") * 2 + lax.axis_index("c")
    base = wid * _BPW
    pltpu.sync_copy(tgt_hbm.at[pl.ds(base, _BPW)], idx_v)
    cp = pltpu.async_copy(lut_hbm.at[idx_v], rows_v, sem)
    pltpu.sync_copy(xt_hbm.at[pl.ds(base, _BPW), :], xv)
    cp.wait()

    def body(p, acc):
        for c in range(_C // 16):
            r = rows_v[p, pl.ds(c * 16, 16)]
            f = xv[p, pl.ds(c * 16, 16)]
            acc = acc + r * f
        return acc

    acc = lax.fori_loop(0, _BPW, body, jnp.zeros((16,), jnp.float32))
    acc_v[...] = acc
    pltpu.sync_copy(acc_v, out_hbm.at[wid])


def kernel(lut, inputs, targets, epoch):
    x = inputs.reshape(2, _C, _NPIX)
    xt = jnp.transpose(x, (0, 2, 1)).reshape(_N_TOT, _C)
    tgt_flat = targets.reshape(_N_TOT)

    lse = _lse_call(lut, x)[0, 0]
    tl_parts = _tgt_dot_kernel(lut, xt, tgt_flat)
    loss = lse - jnp.sum(tl_parts) * (1.0 / _N_TOT)
    return jnp.where(epoch < 0, jnp.float32(0.0), loss)
